# Initial kernel scaffold; baseline (speedup 1.0000x reference)
import jax
import jax.numpy as jnp
from jax.experimental import pallas as pl


def kernel(output_tokens):
    raise NotImplementedError("write your pallas kernel here")


# speedup vs baseline: 2.2865x; 2.2865x over previous
import functools

import jax
import jax.numpy as jnp
from jax import lax
from jax.experimental import pallas as pl
from jax.experimental.pallas import tpu as pltpu
from jax.experimental.pallas import tpu_sc as plsc

_TARGET_LEN = 2048


def _gather_rows_sc(table, flat_idx, num_rows, dim, rows_per_w, chunk):
    info = plsc.get_sparse_core_info()
    nc, ns = info.num_cores, info.num_subcores
    nw = nc * ns
    n_ch = rows_per_w // chunk
    idx3 = flat_idx.reshape(nw, n_ch, chunk)

    mesh = plsc.VectorSubcoreMesh(core_axis_name="c", subcore_axis_name="s")

    @functools.partial(
        pl.kernel,
        out_type=jax.ShapeDtypeStruct((num_rows, dim), jnp.float32),
        mesh=mesh,
        scratch_types=[
            pltpu.VMEM((n_ch, chunk), jnp.int32),
            pltpu.VMEM((2, chunk, dim), jnp.float32),
            pltpu.SemaphoreType.DMA((2,)),
            pltpu.SemaphoreType.DMA((2,)),
        ],
    )
    def body(table_hbm, idx_hbm, out_hbm, idx_v, buf_v, in_sems, out_sems):
        wid = lax.axis_index("s") * nc + lax.axis_index("c")
        base = wid * rows_per_w
        pltpu.sync_copy(idx_hbm.at[wid], idx_v)

        in_d = [None, None]
        out_d = [None, None]

        def issue_gather(g):
            slot = g % 2
            in_d[slot] = pltpu.async_copy(
                table_hbm.at[idx_v.at[g]], buf_v.at[slot], in_sems.at[slot]
            )

        issue_gather(0)
        for g in range(n_ch):
            slot = g % 2
            if g + 1 < n_ch:
                if g >= 1:
                    out_d[(g + 1) % 2].wait()
                issue_gather(g + 1)
            in_d[slot].wait()
            out_d[slot] = pltpu.async_copy(
                buf_v.at[slot],
                out_hbm.at[pl.ds(base + g * chunk, chunk)],
                out_sems.at[slot],
            )
        if n_ch >= 2:
            out_d[(n_ch - 2) % 2].wait()
        out_d[(n_ch - 1) % 2].wait()

    return body(table, idx3)


def kernel(output_tokens):
    batch, seq_len, dim = output_tokens.shape
    idx = jnp.linspace(0.0, seq_len - 1, num=_TARGET_LEN).astype(jnp.int32)
    flat_idx = (
        jnp.arange(batch, dtype=jnp.int32)[:, None] * seq_len + idx[None, :]
    ).reshape(-1)
    table = output_tokens.reshape(batch * seq_len, dim)

    num_rows = batch * _TARGET_LEN
    rows_per_w = num_rows // 32
    chunk = 32

    out = _gather_rows_sc(table, flat_idx, num_rows, dim, rows_per_w, chunk)
    return out.reshape(batch, _TARGET_LEN, dim)
